# baseline (speedup 1.0000x reference)
import functools

import jax
import jax.numpy as jnp
import numpy as np
from jax import lax
from jax.experimental import pallas as pl
from jax.experimental.pallas import tpu as pltpu
from jax.experimental.pallas import tpu_sc as plsc

SR = 8
CH = 128

_G = 8192 // SR
_OFFSETS = np.asarray(
    jax.random.randint(jax.random.key(42), (_G,), 0, SR), dtype=np.int32)


def _tsf_sc(xf, off2d, *, n_batch, seq_len, d):
    g = seq_len // SR
    nch = g // CH
    mesh = plsc.VectorSubcoreMesh(core_axis_name="c", subcore_axis_name="s")
    nc = 2
    nw = 32
    assert n_batch == nw and g % CH == 0
    nbuf = nch - 1

    @functools.partial(
        pl.kernel,
        mesh=mesh,
        out_type=jax.ShapeDtypeStruct((n_batch, g, d), jnp.float32),
        compiler_params=pltpu.CompilerParams(use_tc_tiling_on_sc=True),
        scratch_types=[
            pltpu.VMEM((nch, CH), jnp.int32),
            pltpu.VMEM((nch, CH), jnp.int32),
        ]
        + [pltpu.VMEM((CH, d), jnp.float32) for _ in range(nbuf)]
        + [
            pltpu.SemaphoreType.DMA,
            pltpu.SemaphoreType.DMA,
        ],
    )
    def tsf_kernel(x_hbm, off_hbm, out_hbm, idx_v, off_v, *rest):
        bufs = rest[:nbuf]
        sem_g, sem_o = rest[nbuf], rest[nbuf + 1]
        wid = lax.axis_index("s") * nc + lax.axis_index("c")
        base = wid * seq_len

        pltpu.sync_copy(off_hbm, off_v)
        lane = lax.iota(jnp.int32, 16) * SR

        def fill_idx(c):
            for j in range(CH // 16):
                g0 = c * CH + j * 16
                vals = off_v[c, pl.ds(j * 16, 16)] + (lane + (g0 * SR + base))
                idx_v[c, pl.ds(j * 16, 16)] = vals

        def gather(c, buf):
            return pltpu.async_copy(x_hbm.at[idx_v.at[c]], buf, sem_g)

        gcp = []
        for c in range(nbuf):
            fill_idx(c)
            gcp.append(gather(c, bufs[c]))
        fill_idx(nch - 1)

        ocp = []
        for c in range(nch):
            gcp[c].wait()
            ocp.append(pltpu.async_copy(
                bufs[c % nbuf], out_hbm.at[wid, pl.ds(c * CH, CH)], sem_o))
            if c == 0:
                ocp[0].wait()
                gcp.append(gather(nch - 1, bufs[0]))
        for c in range(1, nch):
            ocp[c].wait()

    return tsf_kernel(xf, off2d)


def kernel(x):
    n_batch, s, d = x.shape
    seq_len = s - s % SR
    g = seq_len // SR
    off2d = jnp.asarray(_OFFSETS).reshape(g // CH, CH)
    xf = x.reshape(n_batch * s, d)
    return _tsf_sc(xf, off2d, n_batch=n_batch, seq_len=seq_len, d=d)


# speedup vs baseline: 1.0010x; 1.0010x over previous
import functools

import jax
import jax.numpy as jnp
import numpy as np
from jax import lax
from jax.experimental import pallas as pl
from jax.experimental.pallas import tpu as pltpu
from jax.experimental.pallas import tpu_sc as plsc

SR = 8
CH = 128

_G = 8192 // SR
_OFFSETS = np.asarray(
    jax.random.randint(jax.random.key(42), (_G,), 0, SR), dtype=np.int32)


def _tsf_sc(xf, off2d, *, n_batch, seq_len, d):
    g = seq_len // SR
    nch = g // CH
    mesh = plsc.VectorSubcoreMesh(core_axis_name="c", subcore_axis_name="s")
    nc = 2
    nw = 32
    assert n_batch == nw and nch == 8
    arena_rows = (nch - 1) * CH

    @functools.partial(
        pl.kernel,
        mesh=mesh,
        out_type=jax.ShapeDtypeStruct((n_batch, g, d), jnp.float32),
        scratch_types=[
            pltpu.VMEM((nch, CH), jnp.int32),
            pltpu.VMEM((nch, CH), jnp.int32),
            pltpu.VMEM((arena_rows, d), jnp.float32),
            pltpu.SemaphoreType.DMA,
            pltpu.SemaphoreType.DMA,
        ],
    )
    def tsf_kernel(x_hbm, off_hbm, out_hbm, idx_v, off_v, arena, sem_g, sem_o):
        wid = lax.axis_index("s") * nc + lax.axis_index("c")
        base = wid * seq_len

        pltpu.sync_copy(off_hbm, off_v)
        lane = lax.iota(jnp.int32, 16) * SR

        def fill_idx(c):
            for j in range(CH // 16):
                g0 = c * CH + j * 16
                vals = off_v[c, pl.ds(j * 16, 16)] + (lane + (g0 * SR + base))
                idx_v[c, pl.ds(j * 16, 16)] = vals

        def gather(c, slot):
            return pltpu.async_copy(
                x_hbm.at[idx_v.at[c]], arena.at[pl.ds(slot * CH, CH)], sem_g)

        def copy_out(row0, nrows, out0):
            return pltpu.async_copy(
                arena.at[pl.ds(row0, nrows)],
                out_hbm.at[wid, pl.ds(out0, nrows)], sem_o)

        gcp = []
        for c in range(nch - 1):
            fill_idx(c)
            gcp.append(gather(c, c))
        fill_idx(nch - 1)

        gcp[0].wait()
        gcp[1].wait()
        o01 = copy_out(0, 2 * CH, 0)
        o01.wait()
        gcp.append(gather(nch - 1, 0))
        gcp[2].wait()
        gcp[3].wait()
        o23 = copy_out(2 * CH, 2 * CH, 2 * CH)
        gcp[4].wait()
        gcp[5].wait()
        o45 = copy_out(4 * CH, 2 * CH, 4 * CH)
        gcp[6].wait()
        o6 = copy_out(6 * CH, CH, 6 * CH)
        gcp[7].wait()
        o7 = copy_out(0, CH, 7 * CH)
        o23.wait()
        o45.wait()
        o6.wait()
        o7.wait()

    return tsf_kernel(xf, off2d)


def kernel(x):
    n_batch, s, d = x.shape
    seq_len = s - s % SR
    g = seq_len // SR
    off2d = jnp.asarray(_OFFSETS).reshape(g // CH, CH)
    xf = x.reshape(n_batch * s, d)
    return _tsf_sc(xf, off2d, n_batch=n_batch, seq_len=seq_len, d=d)


# speedup vs baseline: 1.0054x; 1.0044x over previous
import functools

import jax
import jax.numpy as jnp
import numpy as np
from jax import lax
from jax.experimental import pallas as pl
from jax.experimental.pallas import tpu as pltpu
from jax.experimental.pallas import tpu_sc as plsc

SR = 8
CH = 128


class _Waiter:
    def __init__(self, fn):
        self._fn = fn

    def wait(self):
        self._fn()

_G = 8192 // SR
_OFFSETS = np.asarray(
    jax.random.randint(jax.random.key(42), (_G,), 0, SR), dtype=np.int32)


def _tsf_sc(xf, off2d, *, n_batch, seq_len, d):
    g = seq_len // SR
    nch = g // CH
    mesh = plsc.VectorSubcoreMesh(core_axis_name="c", subcore_axis_name="s")
    nc = 2
    nw = 32
    assert n_batch == nw and nch == 8
    arena_rows = (nch - 1) * CH

    @functools.partial(
        pl.kernel,
        mesh=mesh,
        out_type=jax.ShapeDtypeStruct((n_batch, g, d), jnp.float32),
        scratch_types=[
            pltpu.VMEM((nch, CH), jnp.int32),
            pltpu.VMEM((nch, CH), jnp.int32),
            pltpu.VMEM((arena_rows, d), jnp.float32),
            pltpu.SemaphoreType.DMA,
            pltpu.SemaphoreType.DMA,
        ],
    )
    def tsf_kernel(x_hbm, off_hbm, out_hbm, idx_v, off_v, arena, sem_g, sem_o):
        wid = lax.axis_index("s") * nc + lax.axis_index("c")
        base = wid * seq_len

        pltpu.sync_copy(off_hbm, off_v)
        lane = lax.iota(jnp.int32, 16) * SR

        def fill_idx(c):
            for j in range(CH // 16):
                g0 = c * CH + j * 16
                vals = off_v[c, pl.ds(j * 16, 16)] + (lane + (g0 * SR + base))
                idx_v[c, pl.ds(j * 16, 16)] = vals

        def gather(c, slot):
            return pltpu.async_copy(
                x_hbm.at[idx_v.at[c]], arena.at[pl.ds(slot * CH, CH)], sem_g)

        def copy_out(row0, nrows, out0):
            return pltpu.async_copy(
                arena.at[pl.ds(row0, nrows)],
                out_hbm.at[wid, pl.ds(out0, nrows)], sem_o)

        def fill_and_fire(c, carry):
            for j in range(CH // 16):
                vals = (off_v[c, pl.ds(j * 16, 16)]
                        + (lane + ((c * CH + j * 16) * SR + base)))
                idx_v[c, pl.ds(j * 16, 16)] = vals
            row0 = pl.multiple_of(c * CH, CH)
            pltpu.async_copy(
                x_hbm.at[idx_v.at[c]], arena.at[pl.ds(row0, CH)], sem_g)
            return carry

        lax.fori_loop(0, nch - 1, fill_and_fire, 0, unroll=False)
        fill_idx(nch - 1)

        def gwait():
            pltpu.make_async_copy(
                x_hbm.at[idx_v.at[0]], arena.at[pl.ds(0, CH)], sem_g).wait()

        gcp = [_Waiter(gwait) for _ in range(nch - 1)]

        gcp[0].wait()
        gcp[1].wait()
        o01 = copy_out(0, 2 * CH, 0)
        o01.wait()
        gcp.append(gather(nch - 1, 0))
        gcp[2].wait()
        gcp[3].wait()
        o23 = copy_out(2 * CH, 2 * CH, 2 * CH)
        gcp[4].wait()
        gcp[5].wait()
        o45 = copy_out(4 * CH, 2 * CH, 4 * CH)
        gcp[6].wait()
        o6 = copy_out(6 * CH, CH, 6 * CH)
        gcp[7].wait()
        o7 = copy_out(0, CH, 7 * CH)
        o23.wait()
        o45.wait()
        o6.wait()
        o7.wait()

    return tsf_kernel(xf, off2d)


def kernel(x):
    n_batch, s, d = x.shape
    seq_len = s - s % SR
    g = seq_len // SR
    off2d = jnp.asarray(_OFFSETS).reshape(g // CH, CH)
    xf = x.reshape(n_batch * s, d)
    return _tsf_sc(xf, off2d, n_batch=n_batch, seq_len=seq_len, d=d)
